# baseline (speedup 1.0000x reference)
import functools

import jax
import jax.numpy as jnp
from jax import lax
from jax.experimental import pallas as pl
from jax.experimental.pallas import tpu as pltpu
from jax.experimental.pallas import tpu_sc as plsc

_NC, _NS = 2, 16
_NW = _NC * _NS
_CHUNK = 128
_NBUF = 8
_PF = 4


def _cdiv(a, b):
    return (a + b - 1) // b


def _mesh():
    return plsc.VectorSubcoreMesh(core_axis_name="c", subcore_axis_name="s")


def _make_deg_kernel(n_pad, ch, d):
    rows = n_pad // _NS

    @functools.partial(
        pl.kernel,
        out_type=jax.ShapeDtypeStruct((_NC * n_pad * d,), jnp.float32),
        mesh=_mesh(),
        scratch_types=[
            pltpu.VMEM((ch, _CHUNK), jnp.int32),
            pltpu.VMEM((_CHUNK,), jnp.float32),
            pltpu.VMEM((rows,), jnp.float32),
            pltpu.VMEM((rows * d,), jnp.float32),
            pltpu.VMEM_SHARED((n_pad,), jnp.float32),
        ],
        compiler_params=pltpu.CompilerParams(use_tc_tiling_on_sc=False,
                                             needs_layout_passes=False),
    )
    def deg_kernel(e_hbm, zeros_hbm, out_hbm, idx_v, ones_v, bounce, ebuf,
                   acc):
        c = lax.axis_index("c")
        s = lax.axis_index("s")
        wid = c * _NS + s
        pltpu.sync_copy(zeros_hbm.at[pl.ds(s * rows, rows)], bounce)
        pltpu.sync_copy(bounce, acc.at[pl.ds(s * rows, rows)])
        for i in range(_CHUNK // 16):
            ones_v[pl.ds(i * 16, 16)] = jnp.ones((16,), jnp.float32)
        pltpu.sync_copy(e_hbm.at[1, wid], idx_v)
        plsc.subcore_barrier()

        def body(j, carry):
            pltpu.sync_copy(ones_v, acc.at[idx_v.at[j]], add=True)
            return carry

        lax.fori_loop(0, ch, body, 0)
        plsc.subcore_barrier()
        pltpu.sync_copy(acc.at[pl.ds(s * rows, rows)], bounce)

        def expand(ni, carry):
            v = plsc.load_gather(bounce, [jnp.full((16,), ni, jnp.int32)])
            for k in range(d // 16):
                ebuf[pl.ds(ni * d + k * 16, 16)] = v
            return carry

        lax.fori_loop(0, rows, expand, 0)
        pltpu.sync_copy(
            ebuf, out_hbm.at[pl.ds((c * n_pad + s * rows) * d, rows * d)])

    return deg_kernel


def _make_agg_kernel(n_pad, ch, d):
    rows = n_pad // _NS

    @functools.partial(
        pl.kernel,
        out_type=jax.ShapeDtypeStruct((_NC, n_pad, d), jnp.float32),
        mesh=_mesh(),
        scratch_types=[
            pltpu.VMEM((ch, _CHUNK), jnp.int32),
            pltpu.VMEM((ch, _CHUNK), jnp.int32),
            pltpu.VMEM((_NBUF, _CHUNK, d), jnp.float32),
            pltpu.VMEM((rows, d), jnp.float32),
            [pltpu.SemaphoreType.DMA] * _NBUF,
            [pltpu.SemaphoreType.DMA] * _NBUF,
            pltpu.VMEM_SHARED((n_pad, d), jnp.float32),
        ],
        compiler_params=pltpu.CompilerParams(use_tc_tiling_on_sc=False),
    )
    def agg_kernel(g_hbm, e_hbm, zeros_hbm, out_hbm,
                   src_v, dst_v, rows_v, bounce, sem_g, sem_s, acc):
        c = lax.axis_index("c")
        s = lax.axis_index("s")
        wid = c * _NS + s
        pltpu.sync_copy(zeros_hbm.at[pl.ds(s * rows, rows)], bounce)
        pltpu.sync_copy(bounce, acc.at[pl.ds(s * rows, rows)])
        pltpu.sync_copy(e_hbm.at[0, wid], src_v)
        pltpu.sync_copy(e_hbm.at[1, wid], dst_v)
        plsc.subcore_barrier()

        def wait_on(sem, b):
            pltpu.make_async_copy(g_hbm.at[pl.ds(0, _CHUNK)],
                                  rows_v.at[b], sem).wait()

        for b in range(_PF):
            pltpu.async_copy(g_hbm.at[src_v.at[b]], rows_v.at[b], sem_g[b])

        def outer(jo, carry):
            for b in range(_NBUF):
                j = jo * _NBUF + b
                jn = j + _PF
                bp = (b + _PF) % _NBUF

                @pl.when(jn < ch)
                def _():
                    @pl.when(jn >= _NBUF)
                    def _():
                        wait_on(sem_s[bp], bp)
                    pltpu.async_copy(g_hbm.at[src_v.at[jn]], rows_v.at[bp],
                                     sem_g[bp])

                wait_on(sem_g[b], b)
                pltpu.async_copy(rows_v.at[b], acc.at[dst_v.at[j]], sem_s[b],
                                 add=True)
            return carry

        lax.fori_loop(0, ch // _NBUF, outer, 0)
        for b in range(_NBUF):
            wait_on(sem_s[b], b)
        plsc.subcore_barrier()
        pltpu.sync_copy(acc.at[pl.ds(s * rows, rows)], bounce)
        pltpu.sync_copy(bounce, out_hbm.at[c, pl.ds(s * rows, rows)])

    return agg_kernel


def _make_compact_body(n, n_trash, real_ch, bl, deinterleave):
    def compact_body(x_ref, o_ref):
        c = pl.program_id(0)
        j = pl.program_id(1)
        if deinterleave:
            xin = x_ref[0]
            ev = xin[:, ::2]
            out = jnp.concatenate([ev[::2, :], ev[1::2, :]], axis=1)
        else:
            out = x_ref[0]
        row = lax.broadcasted_iota(jnp.int32, (bl, 128), 0)
        lane = lax.broadcasted_iota(jnp.int32, (bl, 128), 1)
        idx = row * 128 + lane
        padval = jnp.where(c == 1, n + idx % n_trash, idx % n)
        o_ref[0] = jnp.where(row + j * bl >= real_ch, padval, out)
    return compact_body


def _mm_body(x4_ref, wbd_ref, o_ref):
    o_ref[...] = jnp.dot(x4_ref[...], wbd_ref[...],
                         preferred_element_type=jnp.float32)


def _scale_body(xw_ref, d0_ref, d1_ref, o_ref):
    dis = lax.rsqrt(d0_ref[0] + d1_ref[0] + 1.0)
    o_ref[...] = xw_ref[...] * dis


def _fused_mid_body(a0_ref, a1_ref, g_ref, d0_ref, d1_ref, bf_ref, wbd_ref,
                    o_ref):
    dis = lax.rsqrt(d0_ref[0] + d1_ref[0] + 1.0)
    h = jnp.maximum((a0_ref[0] + a1_ref[0] + g_ref[...]) * dis + bf_ref[...],
                    0.0)
    o_ref[...] = jnp.dot(h, wbd_ref[...],
                         preferred_element_type=jnp.float32) * dis


def _fused_out_body(a0_ref, a1_ref, g_ref, d0_ref, d1_ref, bf_ref,
                    wcbd_ref, bcf_ref, o_ref):
    dis = lax.rsqrt(d0_ref[0] + d1_ref[0] + 1.0)
    h = jnp.maximum(
        (a0_ref[0] + a1_ref[0] + g_ref[...]) * dis + bf_ref[...], 0.0)
    o_ref[...] = (jnp.dot(h, wcbd_ref[...],
                          preferred_element_type=jnp.float32)
                  + bcf_ref[...])


def kernel(x, edge_index, W1, b1, W2, b2, Wc, bc):
    n, d_in = x.shape
    e = edge_index.shape[1]
    d = W1.shape[1]
    d_out = Wc.shape[1]

    ch = _cdiv(_cdiv(e, _NW * _CHUNK), _NBUF) * _NBUF
    e_pad = _NW * ch * _CHUNK
    n_pad = _cdiv(n + 1, _NS * 16) * _NS * 16

    rb = 2048
    grid = (_cdiv(n, rb),)
    gr = 128 // d
    nf, fb = n * d // 128, rb * d // 128
    nfp = n_pad * d // 128

    ch_tot = _NW * ch
    bl = 128
    deint = edge_index.dtype == jnp.int64
    if deint:
        e2v = lax.bitcast_convert_type(edge_index, jnp.int32)
        e2v = e2v.reshape(2, e * 2 // 128, 128)
        in_spec = pl.BlockSpec((1, 2 * bl, 128), lambda c, j: (c, j, 0))
    else:
        e2v = edge_index.astype(jnp.int32).reshape(2, e // 128, 128)
        in_spec = pl.BlockSpec((1, bl, 128), lambda c, j: (c, j, 0))
    e_slab = pl.pallas_call(
        _make_compact_body(n, n_pad - n, e // _CHUNK, bl, deint),
        grid=(2, ch_tot // bl),
        in_specs=[in_spec],
        out_specs=pl.BlockSpec((1, bl, 128), lambda c, j: (c, j, 0)),
        out_shape=jax.ShapeDtypeStruct((2, ch_tot, 128), jnp.int32),
    )(e2v).reshape(2, _NW, ch, _CHUNK)
    zeros_deg = jnp.zeros((n_pad,), jnp.float32)
    zeros_big = jnp.zeros((n_pad, d), jnp.float32)

    x4 = x.reshape(n // gr, gr * d_in)
    w1_bd = jnp.kron(jnp.eye(gr, dtype=jnp.float32), W1)
    w2_bd = jnp.kron(jnp.eye(gr, dtype=jnp.float32), W2)
    b1f = jnp.tile(b1, gr).reshape(1, 128)
    b2f = jnp.tile(b2, gr).reshape(1, 128)

    degb = _make_deg_kernel(n_pad, ch, d)(e_slab, zeros_deg)
    degb = degb.reshape(_NC, nfp, 128)

    xwf = pl.pallas_call(
        _mm_body,
        grid=grid,
        in_specs=[
            pl.BlockSpec((rb // gr, gr * d_in), lambda i: (i, 0)),
            pl.BlockSpec((gr * d_in, 128), lambda i: (0, 0)),
        ],
        out_specs=pl.BlockSpec((fb, 128), lambda i: (i, 0)),
        out_shape=jax.ShapeDtypeStruct((nf, 128), jnp.float32),
    )(x4, w1_bd)

    g1f = pl.pallas_call(
        _scale_body,
        grid=grid,
        in_specs=[
            pl.BlockSpec((fb, 128), lambda i: (i, 0)),
            pl.BlockSpec((1, fb, 128), lambda i: (0, i, 0)),
            pl.BlockSpec((1, fb, 128), lambda i: (1, i, 0)),
        ],
        out_specs=pl.BlockSpec((fb, 128), lambda i: (i, 0)),
        out_shape=jax.ShapeDtypeStruct((nf, 128), jnp.float32),
    )(xwf, degb, degb)

    agg = _make_agg_kernel(n_pad, ch, d)
    a1 = agg(g1f.reshape(n, d), e_slab, zeros_big)
    a1f = a1.reshape(_NC, nfp, 128)

    g2f = pl.pallas_call(
        _fused_mid_body,
        grid=grid,
        in_specs=[
            pl.BlockSpec((1, fb, 128), lambda i: (0, i, 0)),
            pl.BlockSpec((1, fb, 128), lambda i: (1, i, 0)),
            pl.BlockSpec((fb, 128), lambda i: (i, 0)),
            pl.BlockSpec((1, fb, 128), lambda i: (0, i, 0)),
            pl.BlockSpec((1, fb, 128), lambda i: (1, i, 0)),
            pl.BlockSpec((1, 128), lambda i: (0, 0)),
            pl.BlockSpec((128, 128), lambda i: (0, 0)),
        ],
        out_specs=pl.BlockSpec((fb, 128), lambda i: (i, 0)),
        out_shape=jax.ShapeDtypeStruct((nf, 128), jnp.float32),
    )(a1f, a1f, g1f, degb, degb, b1f, w2_bd)

    a2 = agg(g2f.reshape(n, d), e_slab, zeros_big)
    a2f = a2.reshape(_NC, nfp, 128)

    wc_bd = jnp.kron(jnp.eye(gr, dtype=jnp.float32), Wc)
    bcf = jnp.tile(bc, gr).reshape(1, gr * d_out)
    yf = pl.pallas_call(
        _fused_out_body,
        grid=grid,
        in_specs=[
            pl.BlockSpec((1, fb, 128), lambda i: (0, i, 0)),
            pl.BlockSpec((1, fb, 128), lambda i: (1, i, 0)),
            pl.BlockSpec((fb, 128), lambda i: (i, 0)),
            pl.BlockSpec((1, fb, 128), lambda i: (0, i, 0)),
            pl.BlockSpec((1, fb, 128), lambda i: (1, i, 0)),
            pl.BlockSpec((1, 128), lambda i: (0, 0)),
            pl.BlockSpec((128, gr * d_out), lambda i: (0, 0)),
            pl.BlockSpec((1, gr * d_out), lambda i: (0, 0)),
        ],
        out_specs=pl.BlockSpec((fb, gr * d_out), lambda i: (i, 0)),
        out_shape=jax.ShapeDtypeStruct((nf, gr * d_out), jnp.float32),
    )(a2f, a2f, g2f, degb, degb, b2f, wc_bd, bcf)

    return yf.reshape(n, d_out)


# speedup vs baseline: 1.0856x; 1.0856x over previous
import functools

import jax
import jax.numpy as jnp
from jax import lax
from jax.experimental import pallas as pl
from jax.experimental.pallas import tpu as pltpu
from jax.experimental.pallas import tpu_sc as plsc

_NC, _NS = 2, 16
_NW = _NC * _NS
_CHUNK = 128
_NBUF = 8
_PF = 4


def _cdiv(a, b):
    return (a + b - 1) // b


def _mesh():
    return plsc.VectorSubcoreMesh(core_axis_name="c", subcore_axis_name="s")


def _make_deg_kernel(n_pad, ch, d):
    rows = n_pad // _NS

    @functools.partial(
        pl.kernel,
        out_type=jax.ShapeDtypeStruct((_NC * n_pad * d,), jnp.float32),
        mesh=_mesh(),
        scratch_types=[
            pltpu.VMEM((ch, _CHUNK), jnp.int32),
            pltpu.VMEM((_CHUNK,), jnp.float32),
            pltpu.VMEM((rows,), jnp.float32),
            pltpu.VMEM((rows * d,), jnp.float32),
            pltpu.VMEM_SHARED((n_pad,), jnp.float32),
        ],
        compiler_params=pltpu.CompilerParams(use_tc_tiling_on_sc=False,
                                             needs_layout_passes=False),
    )
    def deg_kernel(e_hbm, zeros_hbm, out_hbm, idx_v, ones_v, bounce, ebuf,
                   acc):
        c = lax.axis_index("c")
        s = lax.axis_index("s")
        wid = c * _NS + s
        pltpu.sync_copy(zeros_hbm.at[pl.ds(s * rows, rows)], bounce)
        pltpu.sync_copy(bounce, acc.at[pl.ds(s * rows, rows)])
        for i in range(_CHUNK // 16):
            ones_v[pl.ds(i * 16, 16)] = jnp.ones((16,), jnp.float32)
        pltpu.sync_copy(e_hbm.at[1, wid], idx_v)
        plsc.subcore_barrier()

        def body(j, carry):
            pltpu.sync_copy(ones_v, acc.at[idx_v.at[j]], add=True)
            return carry

        lax.fori_loop(0, ch, body, 0)
        plsc.subcore_barrier()
        pltpu.sync_copy(acc.at[pl.ds(s * rows, rows)], bounce)

        def expand(ni, carry):
            v = plsc.load_gather(bounce, [jnp.full((16,), ni, jnp.int32)])
            for k in range(d // 16):
                ebuf[pl.ds(ni * d + k * 16, 16)] = v
            return carry

        lax.fori_loop(0, rows, expand, 0)
        pltpu.sync_copy(
            ebuf, out_hbm.at[pl.ds((c * n_pad + s * rows) * d, rows * d)])

    return deg_kernel


def _make_agg_kernel(n_pad, ch, d):
    rows = n_pad // _NS

    @functools.partial(
        pl.kernel,
        out_type=jax.ShapeDtypeStruct((_NC, n_pad, d), jnp.float32),
        mesh=_mesh(),
        scratch_types=[
            pltpu.VMEM((ch, _CHUNK), jnp.int32),
            pltpu.VMEM((ch, _CHUNK), jnp.int32),
            pltpu.VMEM((_NBUF, _CHUNK, d), jnp.float32),
            pltpu.VMEM((rows, d), jnp.float32),
            [pltpu.SemaphoreType.DMA] * _NBUF,
            [pltpu.SemaphoreType.DMA] * _NBUF,
            pltpu.VMEM_SHARED((n_pad, d), jnp.float32),
        ],
        compiler_params=pltpu.CompilerParams(use_tc_tiling_on_sc=False),
    )
    def agg_kernel(g_hbm, e_hbm, zeros_hbm, out_hbm,
                   src_v, dst_v, rows_v, bounce, sem_g, sem_s, acc):
        c = lax.axis_index("c")
        s = lax.axis_index("s")
        wid = c * _NS + s
        pltpu.sync_copy(zeros_hbm.at[pl.ds(s * rows, rows)], bounce)
        pltpu.sync_copy(bounce, acc.at[pl.ds(s * rows, rows)])
        pltpu.sync_copy(e_hbm.at[0, wid], src_v)
        pltpu.sync_copy(e_hbm.at[1, wid], dst_v)
        plsc.subcore_barrier()

        def wait_on(sem, b):
            pltpu.make_async_copy(g_hbm.at[pl.ds(0, _CHUNK)],
                                  rows_v.at[b], sem).wait()

        for b in range(_PF):
            pltpu.async_copy(g_hbm.at[src_v.at[b]], rows_v.at[b], sem_g[b])

        def outer(jo, carry):
            for b in range(_NBUF):
                j = jo * _NBUF + b
                jn = j + _PF
                bp = (b + _PF) % _NBUF

                @pl.when(jn < ch)
                def _():
                    @pl.when(jn >= _NBUF)
                    def _():
                        wait_on(sem_s[bp], bp)
                    pltpu.async_copy(g_hbm.at[src_v.at[jn]], rows_v.at[bp],
                                     sem_g[bp])

                wait_on(sem_g[b], b)
                pltpu.async_copy(rows_v.at[b], acc.at[dst_v.at[j]], sem_s[b],
                                 add=True)
            return carry

        lax.fori_loop(0, ch // _NBUF, outer, 0)
        for b in range(_NBUF):
            wait_on(sem_s[b], b)
        plsc.subcore_barrier()
        pltpu.sync_copy(acc.at[pl.ds(s * rows, rows)], bounce)
        pltpu.sync_copy(bounce, out_hbm.at[c, pl.ds(s * rows, rows)])

    return agg_kernel


def _mm_body(x4_ref, wbd_ref, o_ref):
    o_ref[...] = jnp.dot(x4_ref[...], wbd_ref[...],
                         preferred_element_type=jnp.float32)


def _scale_body(xw_ref, d0_ref, d1_ref, o_ref):
    dis = lax.rsqrt(d0_ref[0] + d1_ref[0] + 1.0)
    o_ref[...] = xw_ref[...] * dis


def _fused_mid_body(a0_ref, a1_ref, g_ref, d0_ref, d1_ref, bf_ref, wbd_ref,
                    o_ref):
    dis = lax.rsqrt(d0_ref[0] + d1_ref[0] + 1.0)
    h = jnp.maximum((a0_ref[0] + a1_ref[0] + g_ref[...]) * dis + bf_ref[...],
                    0.0)
    o_ref[...] = jnp.dot(h, wbd_ref[...],
                         preferred_element_type=jnp.float32) * dis


def _fused_out_body(a0_ref, a1_ref, g_ref, d0_ref, d1_ref, bf_ref,
                    wcbd_ref, bcf_ref, o_ref):
    dis = lax.rsqrt(d0_ref[0] + d1_ref[0] + 1.0)
    h = jnp.maximum(
        (a0_ref[0] + a1_ref[0] + g_ref[...]) * dis + bf_ref[...], 0.0)
    o_ref[...] = (jnp.dot(h, wcbd_ref[...],
                          preferred_element_type=jnp.float32)
                  + bcf_ref[...])


def kernel(x, edge_index, W1, b1, W2, b2, Wc, bc):
    n, d_in = x.shape
    e = edge_index.shape[1]
    d = W1.shape[1]
    d_out = Wc.shape[1]

    ch = _cdiv(_cdiv(e, _NW * _CHUNK), _NBUF) * _NBUF
    e_pad = _NW * ch * _CHUNK
    n_pad = _cdiv(n + 1, _NS * 16) * _NS * 16

    rb = 2048
    grid = (_cdiv(n, rb),)
    gr = 128 // d
    nf, fb = n * d // 128, rb * d // 128
    nfp = n_pad * d // 128

    e32 = edge_index.astype(jnp.int32)
    pad = e_pad - e
    pad_idx = jnp.arange(pad, dtype=jnp.int32)
    e_slab = jnp.stack([
        jnp.concatenate([e32[0], pad_idx % n]),
        jnp.concatenate([e32[1], n + pad_idx % (n_pad - n)]),
    ]).reshape(2, _NW, ch, _CHUNK)
    zeros_deg = jnp.zeros((n_pad,), jnp.float32)
    zeros_big = jnp.zeros((n_pad, d), jnp.float32)

    x4 = x.reshape(n // gr, gr * d_in)
    w1_bd = jnp.kron(jnp.eye(gr, dtype=jnp.float32), W1)
    w2_bd = jnp.kron(jnp.eye(gr, dtype=jnp.float32), W2)
    b1f = jnp.tile(b1, gr).reshape(1, 128)
    b2f = jnp.tile(b2, gr).reshape(1, 128)

    degb = _make_deg_kernel(n_pad, ch, d)(e_slab, zeros_deg)
    degb = degb.reshape(_NC, nfp, 128)

    xwf = pl.pallas_call(
        _mm_body,
        grid=grid,
        in_specs=[
            pl.BlockSpec((rb // gr, gr * d_in), lambda i: (i, 0)),
            pl.BlockSpec((gr * d_in, 128), lambda i: (0, 0)),
        ],
        out_specs=pl.BlockSpec((fb, 128), lambda i: (i, 0)),
        out_shape=jax.ShapeDtypeStruct((nf, 128), jnp.float32),
    )(x4, w1_bd)

    g1f = pl.pallas_call(
        _scale_body,
        grid=grid,
        in_specs=[
            pl.BlockSpec((fb, 128), lambda i: (i, 0)),
            pl.BlockSpec((1, fb, 128), lambda i: (0, i, 0)),
            pl.BlockSpec((1, fb, 128), lambda i: (1, i, 0)),
        ],
        out_specs=pl.BlockSpec((fb, 128), lambda i: (i, 0)),
        out_shape=jax.ShapeDtypeStruct((nf, 128), jnp.float32),
    )(xwf, degb, degb)

    agg = _make_agg_kernel(n_pad, ch, d)
    a1 = agg(g1f.reshape(n, d), e_slab, zeros_big)
    a1f = a1.reshape(_NC, nfp, 128)

    g2f = pl.pallas_call(
        _fused_mid_body,
        grid=grid,
        in_specs=[
            pl.BlockSpec((1, fb, 128), lambda i: (0, i, 0)),
            pl.BlockSpec((1, fb, 128), lambda i: (1, i, 0)),
            pl.BlockSpec((fb, 128), lambda i: (i, 0)),
            pl.BlockSpec((1, fb, 128), lambda i: (0, i, 0)),
            pl.BlockSpec((1, fb, 128), lambda i: (1, i, 0)),
            pl.BlockSpec((1, 128), lambda i: (0, 0)),
            pl.BlockSpec((128, 128), lambda i: (0, 0)),
        ],
        out_specs=pl.BlockSpec((fb, 128), lambda i: (i, 0)),
        out_shape=jax.ShapeDtypeStruct((nf, 128), jnp.float32),
    )(a1f, a1f, g1f, degb, degb, b1f, w2_bd)

    a2 = agg(g2f.reshape(n, d), e_slab, zeros_big)
    a2f = a2.reshape(_NC, nfp, 128)

    wc_bd = jnp.kron(jnp.eye(gr, dtype=jnp.float32), Wc)
    bcf = jnp.tile(bc, gr).reshape(1, gr * d_out)
    yf = pl.pallas_call(
        _fused_out_body,
        grid=grid,
        in_specs=[
            pl.BlockSpec((1, fb, 128), lambda i: (0, i, 0)),
            pl.BlockSpec((1, fb, 128), lambda i: (1, i, 0)),
            pl.BlockSpec((fb, 128), lambda i: (i, 0)),
            pl.BlockSpec((1, fb, 128), lambda i: (0, i, 0)),
            pl.BlockSpec((1, fb, 128), lambda i: (1, i, 0)),
            pl.BlockSpec((1, 128), lambda i: (0, 0)),
            pl.BlockSpec((128, gr * d_out), lambda i: (0, 0)),
            pl.BlockSpec((1, gr * d_out), lambda i: (0, 0)),
        ],
        out_specs=pl.BlockSpec((fb, gr * d_out), lambda i: (i, 0)),
        out_shape=jax.ShapeDtypeStruct((nf, gr * d_out), jnp.float32),
    )(a2f, a2f, g2f, degb, degb, b2f, wc_bd, bcf)

    return yf.reshape(n, d_out)


# speedup vs baseline: 1.1170x; 1.0289x over previous
import functools

import jax
import jax.numpy as jnp
from jax import lax
from jax.experimental import pallas as pl
from jax.experimental.pallas import tpu as pltpu
from jax.experimental.pallas import tpu_sc as plsc

_NC, _NS = 2, 16
_NW = _NC * _NS
_CHUNK = 128
_NBUF = 8
_PF = 6


def _cdiv(a, b):
    return (a + b - 1) // b


def _mesh():
    return plsc.VectorSubcoreMesh(core_axis_name="c", subcore_axis_name="s")


def _make_deg_kernel(n_pad, ch, d):
    rows = n_pad // _NS

    @functools.partial(
        pl.kernel,
        out_type=jax.ShapeDtypeStruct((_NC * n_pad * d,), jnp.float32),
        mesh=_mesh(),
        scratch_types=[
            pltpu.VMEM((ch, _CHUNK), jnp.int32),
            pltpu.VMEM((_CHUNK,), jnp.float32),
            pltpu.VMEM((rows,), jnp.float32),
            pltpu.VMEM((rows * d,), jnp.float32),
            pltpu.VMEM_SHARED((n_pad,), jnp.float32),
        ],
        compiler_params=pltpu.CompilerParams(use_tc_tiling_on_sc=False,
                                             needs_layout_passes=False),
    )
    def deg_kernel(e_hbm, zeros_hbm, out_hbm, idx_v, ones_v, bounce, ebuf,
                   acc):
        c = lax.axis_index("c")
        s = lax.axis_index("s")
        wid = c * _NS + s
        pltpu.sync_copy(zeros_hbm.at[pl.ds(s * rows, rows)], bounce)
        pltpu.sync_copy(bounce, acc.at[pl.ds(s * rows, rows)])
        for i in range(_CHUNK // 16):
            ones_v[pl.ds(i * 16, 16)] = jnp.ones((16,), jnp.float32)
        pltpu.sync_copy(e_hbm.at[1, wid], idx_v)
        plsc.subcore_barrier()

        def body(j, carry):
            pltpu.sync_copy(ones_v, acc.at[idx_v.at[j]], add=True)
            return carry

        lax.fori_loop(0, ch, body, 0)
        plsc.subcore_barrier()
        pltpu.sync_copy(acc.at[pl.ds(s * rows, rows)], bounce)

        def expand(ni0, carry):
            for k2 in range(16):
                ni = ni0 * 16 + k2
                v = plsc.load_gather(bounce, [jnp.full((16,), ni, jnp.int32)])
                for k in range(d // 16):
                    ebuf[pl.ds(ni * d + k * 16, 16)] = v
            return carry

        lax.fori_loop(0, rows // 16, expand, 0)
        pltpu.sync_copy(
            ebuf, out_hbm.at[pl.ds((c * n_pad + s * rows) * d, rows * d)])

    return deg_kernel


def _make_agg_kernel(n_pad, ch, d):
    rows = n_pad // _NS

    @functools.partial(
        pl.kernel,
        out_type=jax.ShapeDtypeStruct((_NC, n_pad, d), jnp.float32),
        mesh=_mesh(),
        scratch_types=[
            pltpu.VMEM((ch, _CHUNK), jnp.int32),
            pltpu.VMEM((ch, _CHUNK), jnp.int32),
            pltpu.VMEM((_NBUF, _CHUNK, d), jnp.float32),
            pltpu.VMEM((rows, d), jnp.float32),
            [pltpu.SemaphoreType.DMA] * _NBUF,
            [pltpu.SemaphoreType.DMA] * _NBUF,
            pltpu.VMEM_SHARED((n_pad, d), jnp.float32),
        ],
        compiler_params=pltpu.CompilerParams(use_tc_tiling_on_sc=False),
    )
    def agg_kernel(g_hbm, e_hbm, zeros_hbm, out_hbm,
                   src_v, dst_v, rows_v, bounce, sem_g, sem_s, acc):
        c = lax.axis_index("c")
        s = lax.axis_index("s")
        wid = c * _NS + s
        pltpu.sync_copy(zeros_hbm.at[pl.ds(s * rows, rows)], bounce)
        pltpu.sync_copy(bounce, acc.at[pl.ds(s * rows, rows)])
        pltpu.sync_copy(e_hbm.at[0, wid], src_v)
        pltpu.sync_copy(e_hbm.at[1, wid], dst_v)
        plsc.subcore_barrier()

        def wait_on(sem, b):
            pltpu.make_async_copy(g_hbm.at[pl.ds(0, _CHUNK)],
                                  rows_v.at[b], sem).wait()

        for b in range(_PF):
            pltpu.async_copy(g_hbm.at[src_v.at[b]], rows_v.at[b], sem_g[b])

        def outer(jo, carry):
            for b in range(_NBUF):
                j = jo * _NBUF + b
                jn = j + _PF
                bp = (b + _PF) % _NBUF

                @pl.when(jn < ch)
                def _():
                    @pl.when(jn >= _NBUF)
                    def _():
                        wait_on(sem_s[bp], bp)
                    pltpu.async_copy(g_hbm.at[src_v.at[jn]], rows_v.at[bp],
                                     sem_g[bp])

                wait_on(sem_g[b], b)
                pltpu.async_copy(rows_v.at[b], acc.at[dst_v.at[j]], sem_s[b],
                                 add=True)
            return carry

        lax.fori_loop(0, ch // _NBUF, outer, 0)
        for b in range(_NBUF):
            wait_on(sem_s[b], b)
        plsc.subcore_barrier()
        pltpu.sync_copy(acc.at[pl.ds(s * rows, rows)], bounce)
        pltpu.sync_copy(bounce, out_hbm.at[c, pl.ds(s * rows, rows)])

    return agg_kernel


def _mm_body(x4_ref, wbd_ref, o_ref):
    o_ref[...] = jnp.dot(x4_ref[...], wbd_ref[...],
                         preferred_element_type=jnp.float32)


def _scale_body(xw_ref, d0_ref, d1_ref, o_ref):
    dis = lax.rsqrt(d0_ref[0] + d1_ref[0] + 1.0)
    o_ref[...] = xw_ref[...] * dis


def _fused_mid_body(a0_ref, a1_ref, g_ref, d0_ref, d1_ref, bf_ref, wbd_ref,
                    o_ref):
    dis = lax.rsqrt(d0_ref[0] + d1_ref[0] + 1.0)
    h = jnp.maximum((a0_ref[0] + a1_ref[0] + g_ref[...]) * dis + bf_ref[...],
                    0.0)
    o_ref[...] = jnp.dot(h, wbd_ref[...],
                         preferred_element_type=jnp.float32) * dis


def _fused_out_body(a0_ref, a1_ref, g_ref, d0_ref, d1_ref, bf_ref,
                    wcbd_ref, bcf_ref, o_ref):
    dis = lax.rsqrt(d0_ref[0] + d1_ref[0] + 1.0)
    h = jnp.maximum(
        (a0_ref[0] + a1_ref[0] + g_ref[...]) * dis + bf_ref[...], 0.0)
    o_ref[...] = (jnp.dot(h, wcbd_ref[...],
                          preferred_element_type=jnp.float32)
                  + bcf_ref[...])


def kernel(x, edge_index, W1, b1, W2, b2, Wc, bc):
    n, d_in = x.shape
    e = edge_index.shape[1]
    d = W1.shape[1]
    d_out = Wc.shape[1]

    ch = _cdiv(_cdiv(e, _NW * _CHUNK), _NBUF) * _NBUF
    e_pad = _NW * ch * _CHUNK
    n_pad = _cdiv(n + 1, _NS * 16) * _NS * 16

    rb = 2048
    grid = (_cdiv(n, rb),)
    gr = 128 // d
    nf, fb = n * d // 128, rb * d // 128
    nfp = n_pad * d // 128

    e32 = edge_index.astype(jnp.int32)
    pad = e_pad - e
    pad_idx = jnp.arange(pad, dtype=jnp.int32)
    e_slab = jnp.stack([
        jnp.concatenate([e32[0], pad_idx % n]),
        jnp.concatenate([e32[1], n + pad_idx % (n_pad - n)]),
    ]).reshape(2, _NW, ch, _CHUNK)
    zeros_deg = jnp.zeros((n_pad,), jnp.float32)
    zeros_big = jnp.zeros((n_pad, d), jnp.float32)

    x4 = x.reshape(n // gr, gr * d_in)
    w1_bd = jnp.kron(jnp.eye(gr, dtype=jnp.float32), W1)
    w2_bd = jnp.kron(jnp.eye(gr, dtype=jnp.float32), W2)
    b1f = jnp.tile(b1, gr).reshape(1, 128)
    b2f = jnp.tile(b2, gr).reshape(1, 128)

    degb = _make_deg_kernel(n_pad, ch, d)(e_slab, zeros_deg)
    degb = degb.reshape(_NC, nfp, 128)

    xwf = pl.pallas_call(
        _mm_body,
        grid=grid,
        in_specs=[
            pl.BlockSpec((rb // gr, gr * d_in), lambda i: (i, 0)),
            pl.BlockSpec((gr * d_in, 128), lambda i: (0, 0)),
        ],
        out_specs=pl.BlockSpec((fb, 128), lambda i: (i, 0)),
        out_shape=jax.ShapeDtypeStruct((nf, 128), jnp.float32),
    )(x4, w1_bd)

    g1f = pl.pallas_call(
        _scale_body,
        grid=grid,
        in_specs=[
            pl.BlockSpec((fb, 128), lambda i: (i, 0)),
            pl.BlockSpec((1, fb, 128), lambda i: (0, i, 0)),
            pl.BlockSpec((1, fb, 128), lambda i: (1, i, 0)),
        ],
        out_specs=pl.BlockSpec((fb, 128), lambda i: (i, 0)),
        out_shape=jax.ShapeDtypeStruct((nf, 128), jnp.float32),
    )(xwf, degb, degb)

    agg = _make_agg_kernel(n_pad, ch, d)
    a1 = agg(g1f.reshape(n, d), e_slab, zeros_big)
    a1f = a1.reshape(_NC, nfp, 128)

    g2f = pl.pallas_call(
        _fused_mid_body,
        grid=grid,
        in_specs=[
            pl.BlockSpec((1, fb, 128), lambda i: (0, i, 0)),
            pl.BlockSpec((1, fb, 128), lambda i: (1, i, 0)),
            pl.BlockSpec((fb, 128), lambda i: (i, 0)),
            pl.BlockSpec((1, fb, 128), lambda i: (0, i, 0)),
            pl.BlockSpec((1, fb, 128), lambda i: (1, i, 0)),
            pl.BlockSpec((1, 128), lambda i: (0, 0)),
            pl.BlockSpec((128, 128), lambda i: (0, 0)),
        ],
        out_specs=pl.BlockSpec((fb, 128), lambda i: (i, 0)),
        out_shape=jax.ShapeDtypeStruct((nf, 128), jnp.float32),
    )(a1f, a1f, g1f, degb, degb, b1f, w2_bd)

    a2 = agg(g2f.reshape(n, d), e_slab, zeros_big)
    a2f = a2.reshape(_NC, nfp, 128)

    wc_bd = jnp.kron(jnp.eye(gr, dtype=jnp.float32), Wc)
    bcf = jnp.tile(bc, gr).reshape(1, gr * d_out)
    yf = pl.pallas_call(
        _fused_out_body,
        grid=grid,
        in_specs=[
            pl.BlockSpec((1, fb, 128), lambda i: (0, i, 0)),
            pl.BlockSpec((1, fb, 128), lambda i: (1, i, 0)),
            pl.BlockSpec((fb, 128), lambda i: (i, 0)),
            pl.BlockSpec((1, fb, 128), lambda i: (0, i, 0)),
            pl.BlockSpec((1, fb, 128), lambda i: (1, i, 0)),
            pl.BlockSpec((1, 128), lambda i: (0, 0)),
            pl.BlockSpec((128, gr * d_out), lambda i: (0, 0)),
            pl.BlockSpec((1, gr * d_out), lambda i: (0, 0)),
        ],
        out_specs=pl.BlockSpec((fb, gr * d_out), lambda i: (i, 0)),
        out_shape=jax.ShapeDtypeStruct((nf, gr * d_out), jnp.float32),
    )(a2f, a2f, g2f, degb, degb, b2f, wc_bd, bcf)

    return yf.reshape(n, d_out)
